# baseline (speedup 1.0000x reference)
import functools

import jax
import jax.numpy as jnp
from jax import lax
from jax.experimental import pallas as pl
from jax.experimental.pallas import tpu as pltpu
from jax.experimental.pallas import tpu_sc as plsc

N = 10000
D = 128
E = 320000
NC, NS, L = 2, 16, 16
DW = 128
NW = NC * NS
CH = 128
NCHUNK = 80
EPT = CH * NCHUNK
EPAD = EPT * NW
NPAD = 10240
ROWS_PT = NPAD // NS


@functools.lru_cache(maxsize=None)
def _make_sc_segsum():
  mesh = plsc.VectorSubcoreMesh(core_axis_name="c", subcore_axis_name="s")
  out_type = jax.ShapeDtypeStruct((NC * NPAD, D), jnp.float32)
  half = NCHUNK // 2
  scratch = [
      pltpu.VMEM((half, CH), jnp.int32),
      pltpu.VMEM((half, CH), jnp.int32),
      pltpu.VMEM((2 * CH, D), jnp.float32),
      pltpu.SemaphoreType.DMA,
      pltpu.SemaphoreType.DMA,
      pltpu.SemaphoreType.DMA,
      pltpu.SemaphoreType.DMA,
      pltpu.VMEM_SHARED((NPAD, D), jnp.float32),
  ]

  @functools.partial(pl.kernel, mesh=mesh, out_type=out_type,
                     scratch_types=scratch)
  def sc_fn(y_hbm, src_hbm, dst_hbm, z_hbm, acc_out,
            src_v, dst_v, gbuf, sem0, sem1, ssem0, ssem1, acc_sh):
    cid = lax.axis_index("c")
    sid = lax.axis_index("s")
    wid = sid * NC + cid
    g0 = gbuf.at[pl.ds(0, CH)]
    g1 = gbuf.at[pl.ds(CH, CH)]

    pltpu.sync_copy(z_hbm, gbuf)
    for r in range(ROWS_PT // (2 * CH)):
      pltpu.sync_copy(gbuf, acc_sh.at[pl.ds(sid * ROWS_PT + r * 2 * CH,
                                            2 * CH)])

    plsc.subcore_barrier()

    def _pair(i, carry):
      j = 2 * i
      pltpu.make_async_copy(y_hbm.at[src_v.at[j]], g0, sem0).wait()
      pltpu.async_copy(g0, acc_sh.at[dst_v.at[j]], ssem0, add=True)
      pltpu.make_async_copy(y_hbm.at[src_v.at[j + 1]], g1, sem1).wait()
      pltpu.async_copy(g1, acc_sh.at[dst_v.at[j + 1]], ssem1, add=True)
      pltpu.make_async_copy(g0, acc_sh.at[dst_v.at[j]], ssem0).wait()
      pltpu.async_copy(y_hbm.at[src_v.at[j + 2]], g0, sem0)
      pltpu.make_async_copy(g1, acc_sh.at[dst_v.at[j + 1]], ssem1).wait()
      pltpu.async_copy(y_hbm.at[src_v.at[j + 3]], g1, sem1)
      return carry

    for p in range(2):
      pltpu.sync_copy(src_hbm.at[pl.ds(wid * NCHUNK + p * half, half)], src_v)
      pltpu.sync_copy(dst_hbm.at[pl.ds(wid * NCHUNK + p * half, half)], dst_v)
      pltpu.async_copy(y_hbm.at[src_v.at[0]], g0, sem0)
      pltpu.async_copy(y_hbm.at[src_v.at[1]], g1, sem1)
      lax.fori_loop(0, half // 2 - 1, _pair, 0)
      pltpu.make_async_copy(y_hbm.at[src_v.at[half - 2]], g0, sem0).wait()
      pltpu.async_copy(g0, acc_sh.at[dst_v.at[half - 2]], ssem0, add=True)
      pltpu.make_async_copy(y_hbm.at[src_v.at[half - 1]], g1, sem1).wait()
      pltpu.async_copy(g1, acc_sh.at[dst_v.at[half - 1]], ssem1, add=True)
      pltpu.make_async_copy(g0, acc_sh.at[dst_v.at[half - 2]], ssem0).wait()
      pltpu.make_async_copy(g1, acc_sh.at[dst_v.at[half - 1]], ssem1).wait()
    plsc.subcore_barrier()

    base = sid * ROWS_PT
    for r in range(ROWS_PT // (2 * CH)):
      pltpu.sync_copy(acc_sh.at[pl.ds(base + r * 2 * CH, 2 * CH)], gbuf)
      pltpu.sync_copy(gbuf,
                      acc_out.at[pl.ds(cid * NPAD + base + r * 2 * CH,
                                       2 * CH)])

  return sc_fn


@functools.lru_cache(maxsize=None)
def _make_sc_deg():
  mesh = plsc.VectorSubcoreMesh(core_axis_name="c", subcore_axis_name="s")
  out_type = jax.ShapeDtypeStruct((NC * NPAD, DW), jnp.float32)
  scratch = [
      pltpu.VMEM((NCHUNK, CH), jnp.int32),
      pltpu.VMEM((CH, DW), jnp.float32),
      pltpu.VMEM_SHARED((NPAD, DW), jnp.float32),
  ]

  @functools.partial(pl.kernel, mesh=mesh, out_type=out_type,
                     scratch_types=scratch)
  def sc_fn(dst_hbm, zd_hbm, oh_hbm, deg_out, dst_v, oh_v, deg_sh):
    cid = lax.axis_index("c")
    sid = lax.axis_index("s")
    wid = sid * NC + cid

    pltpu.sync_copy(dst_hbm.at[pl.ds(wid * NCHUNK, NCHUNK)], dst_v)
    pltpu.sync_copy(oh_hbm, oh_v)
    for r in range(ROWS_PT // CH):
      pltpu.sync_copy(zd_hbm, deg_sh.at[pl.ds(sid * ROWS_PT + r * CH, CH)])

    plsc.subcore_barrier()

    def _chunk(j, carry):
      pltpu.sync_copy(oh_v, deg_sh.at[dst_v.at[j]], add=True)
      return carry

    lax.fori_loop(0, NCHUNK, _chunk, 0)
    plsc.subcore_barrier()

    base = sid * ROWS_PT
    pltpu.sync_copy(deg_sh.at[pl.ds(base, ROWS_PT)],
                    deg_out.at[pl.ds(cid * NPAD + base, ROWS_PT)])

  return sc_fn


BM = 1000


def _tc_in_body(x_ref, wn_ref, wr_ref, t_ref, r_ref):
  xb = x_ref[...]
  t_ref[...] = jnp.dot(xb, wn_ref[...], preferred_element_type=jnp.float32)
  r_ref[...] = jnp.dot(xb, wr_ref[...], preferred_element_type=jnp.float32)


def _tc_in(x, wn, wr):
  return pl.pallas_call(
      _tc_in_body,
      grid=(N // BM,),
      in_specs=[pl.BlockSpec((BM, D), lambda i: (i, 0)),
                pl.BlockSpec((D, D), lambda i: (0, 0)),
                pl.BlockSpec((D, D), lambda i: (0, 0))],
      out_specs=[pl.BlockSpec((BM, D), lambda i: (i, 0)),
                 pl.BlockSpec((BM, D), lambda i: (i, 0))],
      out_shape=[jax.ShapeDtypeStruct((N, D), jnp.float32),
                 jax.ShapeDtypeStruct((N, D), jnp.float32)],
  )(x, wn, wr)


def _combine(acc_ref, deg_ref):
  agg = acc_ref[0] + acc_ref[1]
  deg = deg_ref[0, :, 0:1] + deg_ref[1, :, 0:1]
  return agg / jnp.maximum(deg, 1.0)


def _tc_mid_body(r1_ref, acc_ref, deg_ref, b_ref, wn_ref, wr_ref,
                 t2_ref, r2_ref):
  mean = _combine(acc_ref, deg_ref)
  h = jnp.maximum(r1_ref[...] + mean + b_ref[...], 0.0)
  t2_ref[...] = jnp.dot(h, wn_ref[...], preferred_element_type=jnp.float32)
  r2_ref[...] = jnp.dot(h, wr_ref[...], preferred_element_type=jnp.float32)


def _tc_mid(r1, acc, deg, b, wn, wr):
  return pl.pallas_call(
      _tc_mid_body,
      grid=(N // BM,),
      in_specs=[pl.BlockSpec((BM, D), lambda i: (i, 0)),
                pl.BlockSpec((NC, BM, D), lambda i: (0, i, 0)),
                pl.BlockSpec((NC, BM, DW), lambda i: (0, i, 0)),
                pl.BlockSpec((1, D), lambda i: (0, 0)),
                pl.BlockSpec((D, D), lambda i: (0, 0)),
                pl.BlockSpec((D, D), lambda i: (0, 0))],
      out_specs=[pl.BlockSpec((BM, D), lambda i: (i, 0)),
                 pl.BlockSpec((BM, D), lambda i: (i, 0))],
      out_shape=[jax.ShapeDtypeStruct((N, D), jnp.float32),
                 jax.ShapeDtypeStruct((N, D), jnp.float32)],
  )(r1, acc, deg, b, wn, wr)


def _tc_out_body(r2_ref, acc_ref, deg_ref, b_ref, o_ref):
  mean = _combine(acc_ref, deg_ref)
  o_ref[...] = r2_ref[...] + mean + b_ref[...]


def _tc_out(r2, acc, deg, b):
  return pl.pallas_call(
      _tc_out_body,
      grid=(N // BM,),
      in_specs=[pl.BlockSpec((BM, D), lambda i: (i, 0)),
                pl.BlockSpec((NC, BM, D), lambda i: (0, i, 0)),
                pl.BlockSpec((NC, BM, DW), lambda i: (0, i, 0)),
                pl.BlockSpec((1, D), lambda i: (0, 0))],
      out_specs=pl.BlockSpec((BM, D), lambda i: (i, 0)),
      out_shape=jax.ShapeDtypeStruct((N, D), jnp.float32),
  )(r2, acc, deg, b)


def kernel(x, edge_index, W1_root, W1_neigh, b1, W2_root, W2_neigh, b2, _):
  src = edge_index[0]
  dst = edge_index[1]
  pad = EPAD - E
  pad_iota = jnp.arange(pad, dtype=jnp.int32)
  src_p = jnp.concatenate(
      [src, pad_iota % N]).reshape(EPAD // CH, CH)
  dst_p = jnp.concatenate(
      [dst, N + pad_iota % (NPAD - N)]).reshape(EPAD // CH, CH)

  z_rows = jnp.zeros((2 * CH, D), jnp.float32)
  zd_rows = jnp.zeros((CH, DW), jnp.float32)
  oh_rows = jnp.tile(
      (jnp.arange(DW) < 1).astype(jnp.float32)[None, :], (CH, 1))

  degp = _make_sc_deg()(dst_p, zd_rows, oh_rows)
  t1, r1 = _tc_in(x, W1_neigh, W1_root)
  acc1 = _make_sc_segsum()(t1, src_p, dst_p, z_rows)
  acc1 = acc1.reshape(NC, NPAD, D)
  degp = degp.reshape(NC, NPAD, DW)
  t2, r2 = _tc_mid(r1, acc1, degp, b1.reshape(1, D), W2_neigh, W2_root)
  acc2 = _make_sc_segsum()(t2, src_p, dst_p, z_rows)
  acc2 = acc2.reshape(NC, NPAD, D)
  out = _tc_out(r2, acc2, degp, b2.reshape(1, D))
  return (out, None)


# speedup vs baseline: 1.1842x; 1.1842x over previous
import functools

import jax
import jax.numpy as jnp
from jax import lax
from jax.experimental import pallas as pl
from jax.experimental.pallas import tpu as pltpu
from jax.experimental.pallas import tpu_sc as plsc

N = 10000
D = 128
E = 320000
NC, NS, L = 2, 16, 16
DW = 128
NW = NC * NS
CH = 128
NCHUNK = 80
EPT = CH * NCHUNK
EPAD = EPT * NW
NPAD = 10240
ROWS_PT = NPAD // NS


@functools.lru_cache(maxsize=None)
def _make_sc_segsum():
  mesh = plsc.VectorSubcoreMesh(core_axis_name="c", subcore_axis_name="s")
  out_type = jax.ShapeDtypeStruct((NC * NPAD, D), jnp.float32)
  half = NCHUNK // 2
  scratch = [
      pltpu.VMEM((half, CH), jnp.int32),
      pltpu.VMEM((half, CH), jnp.int32),
      pltpu.VMEM((2 * CH, D), jnp.float32),
      pltpu.SemaphoreType.DMA,
      pltpu.SemaphoreType.DMA,
      pltpu.VMEM_SHARED((NPAD, D), jnp.float32),
  ]

  @functools.partial(pl.kernel, mesh=mesh, out_type=out_type,
                     scratch_types=scratch)
  def sc_fn(y_hbm, src_hbm, dst_hbm, z_hbm, acc_out,
            src_v, dst_v, gbuf, sem0, sem1, acc_sh):
    cid = lax.axis_index("c")
    sid = lax.axis_index("s")
    wid = sid * NC + cid
    g0 = gbuf.at[pl.ds(0, CH)]
    g1 = gbuf.at[pl.ds(CH, CH)]

    nzb = ROWS_PT // (2 * CH)
    rem = ROWS_PT - nzb * 2 * CH
    pltpu.sync_copy(z_hbm, gbuf)
    for r in range(nzb):
      pltpu.sync_copy(gbuf, acc_sh.at[pl.ds(sid * ROWS_PT + r * 2 * CH,
                                            2 * CH)])
    if rem:
      pltpu.sync_copy(g0, acc_sh.at[pl.ds(sid * ROWS_PT + nzb * 2 * CH, rem)])

    plsc.subcore_barrier()

    def _pair(i, carry):
      j = 2 * i
      pltpu.make_async_copy(y_hbm.at[src_v.at[j]], g0, sem0).wait()
      pltpu.sync_copy(g0, acc_sh.at[dst_v.at[j]], add=True)
      pltpu.async_copy(y_hbm.at[src_v.at[j + 2]], g0, sem0)
      pltpu.make_async_copy(y_hbm.at[src_v.at[j + 1]], g1, sem1).wait()
      pltpu.sync_copy(g1, acc_sh.at[dst_v.at[j + 1]], add=True)
      pltpu.async_copy(y_hbm.at[src_v.at[j + 3]], g1, sem1)
      return carry

    for p in range(2):
      pltpu.sync_copy(src_hbm.at[pl.ds(wid * NCHUNK + p * half, half)], src_v)
      pltpu.sync_copy(dst_hbm.at[pl.ds(wid * NCHUNK + p * half, half)], dst_v)
      pltpu.async_copy(y_hbm.at[src_v.at[0]], g0, sem0)
      pltpu.async_copy(y_hbm.at[src_v.at[1]], g1, sem1)
      lax.fori_loop(0, half // 2 - 1, _pair, 0)
      pltpu.make_async_copy(y_hbm.at[src_v.at[half - 2]], g0, sem0).wait()
      pltpu.sync_copy(g0, acc_sh.at[dst_v.at[half - 2]], add=True)
      pltpu.make_async_copy(y_hbm.at[src_v.at[half - 1]], g1, sem1).wait()
      pltpu.sync_copy(g1, acc_sh.at[dst_v.at[half - 1]], add=True)
    plsc.subcore_barrier()

    base = sid * ROWS_PT
    for r in range(nzb):
      pltpu.sync_copy(acc_sh.at[pl.ds(base + r * 2 * CH, 2 * CH)], gbuf)
      pltpu.sync_copy(gbuf,
                      acc_out.at[pl.ds(cid * NPAD + base + r * 2 * CH,
                                       2 * CH)])
    if rem:
      pltpu.sync_copy(acc_sh.at[pl.ds(base + nzb * 2 * CH, rem)], g0)
      pltpu.sync_copy(g0,
                      acc_out.at[pl.ds(cid * NPAD + base + nzb * 2 * CH, rem)])

  return sc_fn


@functools.lru_cache(maxsize=None)
def _make_sc_deg():
  mesh = plsc.VectorSubcoreMesh(core_axis_name="c", subcore_axis_name="s")
  out_type = jax.ShapeDtypeStruct((NC * NPAD, DW), jnp.float32)
  scratch = [
      pltpu.VMEM((NCHUNK, CH), jnp.int32),
      pltpu.VMEM((CH, DW), jnp.float32),
      pltpu.VMEM_SHARED((NPAD, DW), jnp.float32),
  ]

  @functools.partial(pl.kernel, mesh=mesh, out_type=out_type,
                     scratch_types=scratch)
  def sc_fn(dst_hbm, zd_hbm, oh_hbm, deg_out, dst_v, oh_v, deg_sh):
    cid = lax.axis_index("c")
    sid = lax.axis_index("s")
    wid = sid * NC + cid

    pltpu.sync_copy(dst_hbm.at[pl.ds(wid * NCHUNK, NCHUNK)], dst_v)
    pltpu.sync_copy(oh_hbm, oh_v)
    for r in range(ROWS_PT // CH):
      pltpu.sync_copy(zd_hbm, deg_sh.at[pl.ds(sid * ROWS_PT + r * CH, CH)])

    plsc.subcore_barrier()

    def _chunk(j, carry):
      pltpu.sync_copy(oh_v, deg_sh.at[dst_v.at[j]], add=True)
      return carry

    lax.fori_loop(0, NCHUNK, _chunk, 0)
    plsc.subcore_barrier()

    base = sid * ROWS_PT
    pltpu.sync_copy(deg_sh.at[pl.ds(base, ROWS_PT)],
                    deg_out.at[pl.ds(cid * NPAD + base, ROWS_PT)])

  return sc_fn


BM = 1000


def _tc_in_body(x_ref, wn_ref, wr_ref, t_ref, r_ref):
  xb = x_ref[...]
  t_ref[...] = jnp.dot(xb, wn_ref[...], preferred_element_type=jnp.float32)
  r_ref[...] = jnp.dot(xb, wr_ref[...], preferred_element_type=jnp.float32)


def _tc_in(x, wn, wr):
  return pl.pallas_call(
      _tc_in_body,
      grid=(N // BM,),
      in_specs=[pl.BlockSpec((BM, D), lambda i: (i, 0)),
                pl.BlockSpec((D, D), lambda i: (0, 0)),
                pl.BlockSpec((D, D), lambda i: (0, 0))],
      out_specs=[pl.BlockSpec((BM, D), lambda i: (i, 0)),
                 pl.BlockSpec((BM, D), lambda i: (i, 0))],
      out_shape=[jax.ShapeDtypeStruct((N, D), jnp.float32),
                 jax.ShapeDtypeStruct((N, D), jnp.float32)],
  )(x, wn, wr)


def _combine(acc_ref, deg_ref):
  agg = acc_ref[0] + acc_ref[1]
  deg = deg_ref[0, :, 0:1] + deg_ref[1, :, 0:1]
  return agg / jnp.maximum(deg, 1.0)


def _tc_mid_body(r1_ref, acc_ref, deg_ref, b_ref, wn_ref, wr_ref,
                 t2_ref, r2_ref):
  mean = _combine(acc_ref, deg_ref)
  h = jnp.maximum(r1_ref[...] + mean + b_ref[...], 0.0)
  t2_ref[...] = jnp.dot(h, wn_ref[...], preferred_element_type=jnp.float32)
  r2_ref[...] = jnp.dot(h, wr_ref[...], preferred_element_type=jnp.float32)


def _tc_mid(r1, acc, deg, b, wn, wr):
  return pl.pallas_call(
      _tc_mid_body,
      grid=(N // BM,),
      in_specs=[pl.BlockSpec((BM, D), lambda i: (i, 0)),
                pl.BlockSpec((NC, BM, D), lambda i: (0, i, 0)),
                pl.BlockSpec((NC, BM, DW), lambda i: (0, i, 0)),
                pl.BlockSpec((1, D), lambda i: (0, 0)),
                pl.BlockSpec((D, D), lambda i: (0, 0)),
                pl.BlockSpec((D, D), lambda i: (0, 0))],
      out_specs=[pl.BlockSpec((BM, D), lambda i: (i, 0)),
                 pl.BlockSpec((BM, D), lambda i: (i, 0))],
      out_shape=[jax.ShapeDtypeStruct((N, D), jnp.float32),
                 jax.ShapeDtypeStruct((N, D), jnp.float32)],
  )(r1, acc, deg, b, wn, wr)


def _tc_out_body(r2_ref, acc_ref, deg_ref, b_ref, o_ref):
  mean = _combine(acc_ref, deg_ref)
  o_ref[...] = r2_ref[...] + mean + b_ref[...]


def _tc_out(r2, acc, deg, b):
  return pl.pallas_call(
      _tc_out_body,
      grid=(N // BM,),
      in_specs=[pl.BlockSpec((BM, D), lambda i: (i, 0)),
                pl.BlockSpec((NC, BM, D), lambda i: (0, i, 0)),
                pl.BlockSpec((NC, BM, DW), lambda i: (0, i, 0)),
                pl.BlockSpec((1, D), lambda i: (0, 0))],
      out_specs=pl.BlockSpec((BM, D), lambda i: (i, 0)),
      out_shape=jax.ShapeDtypeStruct((N, D), jnp.float32),
  )(r2, acc, deg, b)


def kernel(x, edge_index, W1_root, W1_neigh, b1, W2_root, W2_neigh, b2, _):
  src = edge_index[0]
  dst = edge_index[1]
  pad = EPAD - E
  pad_iota = jnp.arange(pad, dtype=jnp.int32)
  src_p = jnp.concatenate(
      [src, pad_iota % N]).reshape(EPAD // CH, CH)
  dst_p = jnp.concatenate(
      [dst, N + pad_iota % (NPAD - N)]).reshape(EPAD // CH, CH)

  z_rows = jnp.zeros((2 * CH, D), jnp.float32)
  zd_rows = jnp.zeros((CH, DW), jnp.float32)
  oh_rows = jnp.tile(
      (jnp.arange(DW) < 1).astype(jnp.float32)[None, :], (CH, 1))

  degp = _make_sc_deg()(dst_p, zd_rows, oh_rows)
  t1, r1 = _tc_in(x, W1_neigh, W1_root)
  acc1 = _make_sc_segsum()(t1, src_p, dst_p, z_rows)
  acc1 = acc1.reshape(NC, NPAD, D)
  degp = degp.reshape(NC, NPAD, DW)
  t2, r2 = _tc_mid(r1, acc1, degp, b1.reshape(1, D), W2_neigh, W2_root)
  acc2 = _make_sc_segsum()(t2, src_p, dst_p, z_rows)
  acc2 = acc2.reshape(NC, NPAD, D)
  out = _tc_out(r2, acc2, degp, b2.reshape(1, D))
  return (out, None)


# speedup vs baseline: 1.2073x; 1.0195x over previous
import functools

import jax
import jax.numpy as jnp
from jax import lax
from jax.experimental import pallas as pl
from jax.experimental.pallas import tpu as pltpu
from jax.experimental.pallas import tpu_sc as plsc

N = 10000
D = 128
E = 320000
NC, NS, L = 2, 16, 16
DW = 128
NW = NC * NS
CH = 128
NCHUNK = 80
EPT = CH * NCHUNK
EPAD = EPT * NW
NPAD = 10240
ROWS_PT = NPAD // NS


@functools.lru_cache(maxsize=None)
def _make_sc_segsum():
  mesh = plsc.VectorSubcoreMesh(core_axis_name="c", subcore_axis_name="s")
  out_type = jax.ShapeDtypeStruct((NC * NPAD, D), jnp.float32)
  half = NCHUNK // 2
  scratch = [
      pltpu.VMEM((half, CH), jnp.int32),
      pltpu.VMEM((half, CH), jnp.int32),
      pltpu.VMEM((2 * CH, D), jnp.float32),
      pltpu.SemaphoreType.DMA,
      pltpu.SemaphoreType.DMA,
      pltpu.VMEM_SHARED((NPAD, D), jnp.float32),
  ]

  @functools.partial(pl.kernel, mesh=mesh, out_type=out_type,
                     scratch_types=scratch)
  def sc_fn(y_hbm, src_hbm, dst_hbm, z_hbm, acc_out,
            src_v, dst_v, gbuf, sem0, sem1, acc_sh):
    cid = lax.axis_index("c")
    sid = lax.axis_index("s")
    wid = sid * NC + cid
    g0 = gbuf.at[pl.ds(0, CH)]
    g1 = gbuf.at[pl.ds(CH, CH)]

    nzb = ROWS_PT // (2 * CH)
    rem = ROWS_PT - nzb * 2 * CH
    pltpu.sync_copy(z_hbm, gbuf)
    for r in range(nzb):
      pltpu.sync_copy(gbuf, acc_sh.at[pl.ds(sid * ROWS_PT + r * 2 * CH,
                                            2 * CH)])
    if rem:
      pltpu.sync_copy(g0, acc_sh.at[pl.ds(sid * ROWS_PT + nzb * 2 * CH, rem)])

    plsc.subcore_barrier()

    def _pair(i, carry):
      j = 2 * i
      pltpu.make_async_copy(y_hbm.at[src_v.at[j]], g0, sem0).wait()
      pltpu.sync_copy(g0, acc_sh.at[dst_v.at[j]], add=True)
      pltpu.async_copy(y_hbm.at[src_v.at[j + 2]], g0, sem0)
      pltpu.make_async_copy(y_hbm.at[src_v.at[j + 1]], g1, sem1).wait()
      pltpu.sync_copy(g1, acc_sh.at[dst_v.at[j + 1]], add=True)
      pltpu.async_copy(y_hbm.at[src_v.at[j + 3]], g1, sem1)
      return carry

    for p in range(2):
      pltpu.sync_copy(src_hbm.at[pl.ds(wid * NCHUNK + p * half, half)], src_v)
      pltpu.sync_copy(dst_hbm.at[pl.ds(wid * NCHUNK + p * half, half)], dst_v)
      pltpu.async_copy(y_hbm.at[src_v.at[0]], g0, sem0)
      pltpu.async_copy(y_hbm.at[src_v.at[1]], g1, sem1)
      lax.fori_loop(0, half // 2 - 1, _pair, 0)
      pltpu.make_async_copy(y_hbm.at[src_v.at[half - 2]], g0, sem0).wait()
      pltpu.sync_copy(g0, acc_sh.at[dst_v.at[half - 2]], add=True)
      pltpu.make_async_copy(y_hbm.at[src_v.at[half - 1]], g1, sem1).wait()
      pltpu.sync_copy(g1, acc_sh.at[dst_v.at[half - 1]], add=True)
    plsc.subcore_barrier()

    base = sid * ROWS_PT
    for r in range(nzb):
      pltpu.sync_copy(acc_sh.at[pl.ds(base + r * 2 * CH, 2 * CH)], gbuf)
      pltpu.sync_copy(gbuf,
                      acc_out.at[pl.ds(cid * NPAD + base + r * 2 * CH,
                                       2 * CH)])
    if rem:
      pltpu.sync_copy(acc_sh.at[pl.ds(base + nzb * 2 * CH, rem)], g0)
      pltpu.sync_copy(g0,
                      acc_out.at[pl.ds(cid * NPAD + base + nzb * 2 * CH, rem)])

  return sc_fn


@functools.lru_cache(maxsize=None)
def _make_sc_deg():
  mesh = plsc.VectorSubcoreMesh(core_axis_name="c", subcore_axis_name="s")
  out_type = jax.ShapeDtypeStruct((NC * NPAD, DW), jnp.float32)
  scratch = [
      pltpu.VMEM((NCHUNK, CH), jnp.int32),
      pltpu.VMEM((CH, DW), jnp.float32),
      pltpu.VMEM_SHARED((NPAD, DW), jnp.float32),
  ]

  @functools.partial(pl.kernel, mesh=mesh, out_type=out_type,
                     scratch_types=scratch)
  def sc_fn(dst_hbm, zd_hbm, oh_hbm, deg_out, dst_v, oh_v, deg_sh):
    cid = lax.axis_index("c")
    sid = lax.axis_index("s")
    wid = sid * NC + cid

    pltpu.sync_copy(dst_hbm.at[pl.ds(wid * NCHUNK, NCHUNK)], dst_v)
    pltpu.sync_copy(oh_hbm, oh_v)
    for r in range(ROWS_PT // CH):
      pltpu.sync_copy(zd_hbm, deg_sh.at[pl.ds(sid * ROWS_PT + r * CH, CH)])

    plsc.subcore_barrier()

    def _chunk(j, carry):
      pltpu.sync_copy(oh_v, deg_sh.at[dst_v.at[j]], add=True)
      return carry

    lax.fori_loop(0, NCHUNK, _chunk, 0)
    plsc.subcore_barrier()

    base = sid * ROWS_PT
    pltpu.sync_copy(deg_sh.at[pl.ds(base, ROWS_PT)],
                    deg_out.at[pl.ds(cid * NPAD + base, ROWS_PT)])

  return sc_fn


BM = 2000


def _tc_in_body(x_ref, wn_ref, wr_ref, t_ref, r_ref):
  xb = x_ref[...]
  t_ref[...] = jnp.dot(xb, wn_ref[...], preferred_element_type=jnp.float32)
  r_ref[...] = jnp.dot(xb, wr_ref[...], preferred_element_type=jnp.float32)


def _tc_in(x, wn, wr):
  return pl.pallas_call(
      _tc_in_body,
      grid=(N // BM,),
      in_specs=[pl.BlockSpec((BM, D), lambda i: (i, 0)),
                pl.BlockSpec((D, D), lambda i: (0, 0)),
                pl.BlockSpec((D, D), lambda i: (0, 0))],
      out_specs=[pl.BlockSpec((BM, D), lambda i: (i, 0)),
                 pl.BlockSpec((BM, D), lambda i: (i, 0))],
      out_shape=[jax.ShapeDtypeStruct((N, D), jnp.float32),
                 jax.ShapeDtypeStruct((N, D), jnp.float32)],
  )(x, wn, wr)


def _combine(acc_ref, deg_ref):
  agg = acc_ref[0] + acc_ref[1]
  deg = deg_ref[0, :, 0:1] + deg_ref[1, :, 0:1]
  return agg / jnp.maximum(deg, 1.0)


def _tc_mid_body(r1_ref, acc_ref, deg_ref, b_ref, wn_ref, wr_ref,
                 t2_ref, r2_ref):
  mean = _combine(acc_ref, deg_ref)
  h = jnp.maximum(r1_ref[...] + mean + b_ref[...], 0.0)
  t2_ref[...] = jnp.dot(h, wn_ref[...], preferred_element_type=jnp.float32)
  r2_ref[...] = jnp.dot(h, wr_ref[...], preferred_element_type=jnp.float32)


def _tc_mid(r1, acc, deg, b, wn, wr):
  return pl.pallas_call(
      _tc_mid_body,
      grid=(N // BM,),
      in_specs=[pl.BlockSpec((BM, D), lambda i: (i, 0)),
                pl.BlockSpec((NC, BM, D), lambda i: (0, i, 0)),
                pl.BlockSpec((NC, BM, DW), lambda i: (0, i, 0)),
                pl.BlockSpec((1, D), lambda i: (0, 0)),
                pl.BlockSpec((D, D), lambda i: (0, 0)),
                pl.BlockSpec((D, D), lambda i: (0, 0))],
      out_specs=[pl.BlockSpec((BM, D), lambda i: (i, 0)),
                 pl.BlockSpec((BM, D), lambda i: (i, 0))],
      out_shape=[jax.ShapeDtypeStruct((N, D), jnp.float32),
                 jax.ShapeDtypeStruct((N, D), jnp.float32)],
  )(r1, acc, deg, b, wn, wr)


def _tc_out_body(r2_ref, acc_ref, deg_ref, b_ref, o_ref):
  mean = _combine(acc_ref, deg_ref)
  o_ref[...] = r2_ref[...] + mean + b_ref[...]


def _tc_out(r2, acc, deg, b):
  return pl.pallas_call(
      _tc_out_body,
      grid=(N // BM,),
      in_specs=[pl.BlockSpec((BM, D), lambda i: (i, 0)),
                pl.BlockSpec((NC, BM, D), lambda i: (0, i, 0)),
                pl.BlockSpec((NC, BM, DW), lambda i: (0, i, 0)),
                pl.BlockSpec((1, D), lambda i: (0, 0))],
      out_specs=pl.BlockSpec((BM, D), lambda i: (i, 0)),
      out_shape=jax.ShapeDtypeStruct((N, D), jnp.float32),
  )(r2, acc, deg, b)


def kernel(x, edge_index, W1_root, W1_neigh, b1, W2_root, W2_neigh, b2, _):
  src = edge_index[0]
  dst = edge_index[1]
  pad = EPAD - E
  pad_iota = jnp.arange(pad, dtype=jnp.int32)
  src_p = jnp.concatenate(
      [src, pad_iota % N]).reshape(EPAD // CH, CH)
  dst_p = jnp.concatenate(
      [dst, N + pad_iota % (NPAD - N)]).reshape(EPAD // CH, CH)

  z_rows = jnp.zeros((2 * CH, D), jnp.float32)
  zd_rows = jnp.zeros((CH, DW), jnp.float32)
  oh_rows = jnp.tile(
      (jnp.arange(DW) < 1).astype(jnp.float32)[None, :], (CH, 1))

  degp = _make_sc_deg()(dst_p, zd_rows, oh_rows)
  t1, r1 = _tc_in(x, W1_neigh, W1_root)
  acc1 = _make_sc_segsum()(t1, src_p, dst_p, z_rows)
  acc1 = acc1.reshape(NC, NPAD, D)
  degp = degp.reshape(NC, NPAD, DW)
  t2, r2 = _tc_mid(r1, acc1, degp, b1.reshape(1, D), W2_neigh, W2_root)
  acc2 = _make_sc_segsum()(t2, src_p, dst_p, z_rows)
  acc2 = acc2.reshape(NC, NPAD, D)
  out = _tc_out(r2, acc2, degp, b2.reshape(1, D))
  return (out, None)
